# Initial kernel scaffold; baseline (speedup 1.0000x reference)
import functools

import jax
import jax.numpy as jnp
from jax.experimental import pallas as pl
from jax.experimental.pallas import tpu as pltpu

EMB = 64
MAXN = 50


def _attn(nm, no, q, mask):
    scores = jnp.sum(nm * q[:, None, :], axis=-1)
    scores = jnp.where(mask, scores, jnp.finfo(scores.dtype).min)
    m = jnp.max(scores, axis=1, keepdims=True)
    e = jnp.exp(scores - m)
    p = e / jnp.sum(e, axis=1, keepdims=True)
    return jnp.sum(no * p[:, :, None], axis=1)


def _branch(u, v, nm, no, mask, hop_wT, hop_b, dense_wT, dense_b, out_w):
    q = u + v
    o0 = _attn(nm, no, q, mask)
    q1 = jax.nn.relu(
        jnp.dot(q, hop_wT, preferred_element_type=jnp.float32) + o0 + hop_b)
    o1 = _attn(nm, no, q1, mask)
    x = jnp.concatenate([u * v, o1], axis=1)
    h = jax.nn.relu(
        jnp.dot(x, dense_wT, preferred_element_type=jnp.float32) + dense_b)
    return jnp.sum(h * out_w, axis=1, keepdims=True)


def _cmn_kernel(u_ref, v_ref, vn_ref, nm_ref, no_ref, nmn_ref, non_ref,
                len_ref, lenn_ref, hop_wT_ref, hop_b_ref, dense_wT_ref,
                dense_b_ref, out_w_ref, pos_ref, neg_ref):
    u = u_ref[...]
    hop_wT = hop_wT_ref[...]
    hop_b = hop_b_ref[...]
    dense_wT = dense_wT_ref[...]
    dense_b = dense_b_ref[...]
    out_w = out_w_ref[...]
    bb = u.shape[0]
    pos_iota = jax.lax.broadcasted_iota(jnp.int32, (bb, MAXN), 1)
    mask = pos_iota < len_ref[...]
    mask_n = pos_iota < lenn_ref[...]
    pos_ref[...] = _branch(u, v_ref[...], nm_ref[...], no_ref[...], mask,
                           hop_wT, hop_b, dense_wT, dense_b, out_w)
    neg_ref[...] = _branch(u, vn_ref[...], nmn_ref[...], non_ref[...], mask_n,
                           hop_wT, hop_b, dense_wT, dense_b, out_w)


@functools.partial(jax.jit, static_argnames=("interpret",))
def _cmn_compute(cur_user, cur_item, cur_item_neg, nm, no, nmn, non,
                 lengths, lengths_n, hop_w, hop_b, dense_w, dense_b, out_w,
                 interpret=False):
    B = cur_user.shape[0]
    Bb = min(256, B)
    grid = (B // Bb,)
    row_spec = pl.BlockSpec((Bb, EMB), lambda i: (i, 0))
    neigh_spec = pl.BlockSpec((Bb, MAXN, EMB), lambda i: (i, 0, 0))
    len_spec = pl.BlockSpec((Bb, 1), lambda i: (i, 0))
    w_spec = pl.BlockSpec((EMB, EMB), lambda i: (0, 0))
    dw_spec = pl.BlockSpec((2 * EMB, EMB), lambda i: (0, 0))
    b_spec = pl.BlockSpec((1, EMB), lambda i: (0, 0))
    out_spec = pl.BlockSpec((Bb, 1), lambda i: (i, 0))
    pos, neg = pl.pallas_call(
        _cmn_kernel,
        grid=grid,
        in_specs=[row_spec, row_spec, row_spec,
                  neigh_spec, neigh_spec, neigh_spec, neigh_spec,
                  len_spec, len_spec, w_spec, b_spec, dw_spec, b_spec, b_spec],
        out_specs=[out_spec, out_spec],
        out_shape=[jax.ShapeDtypeStruct((B, 1), jnp.float32),
                   jax.ShapeDtypeStruct((B, 1), jnp.float32)],
        interpret=interpret,
    )(cur_user, cur_item, cur_item_neg, nm, no, nmn, non,
      lengths.reshape(B, 1), lengths_n.reshape(B, 1),
      hop_w.T, hop_b.reshape(1, EMB), dense_w.T, dense_b.reshape(1, EMB),
      out_w)
    return pos[:, 0], neg[:, 0]


def kernel(input_users, input_items, input_items_negative, input_neighborhoods,
           input_neighborhood_lengths, input_neighborhoods_negative,
           input_neighborhood_lengths_negative, user_memory, item_memory,
           user_output, hop_w, hop_b, dense_w, dense_b, out_w):
    cur_user = jnp.take(user_memory, input_users, axis=0)
    cur_item = jnp.take(item_memory, input_items, axis=0)
    cur_item_neg = jnp.take(item_memory, input_items_negative, axis=0)
    nm = jnp.take(user_memory, input_neighborhoods, axis=0)
    no = jnp.take(user_output, input_neighborhoods, axis=0)
    nmn = jnp.take(user_memory, input_neighborhoods_negative, axis=0)
    non = jnp.take(user_output, input_neighborhoods_negative, axis=0)
    return _cmn_compute(cur_user, cur_item, cur_item_neg, nm, no, nmn, non,
                        input_neighborhood_lengths,
                        input_neighborhood_lengths_negative,
                        hop_w, hop_b, dense_w, dense_b, out_w)


# speedup vs baseline: 1.4973x; 1.4973x over previous
import functools

import jax
import jax.numpy as jnp
from jax.experimental import pallas as pl
from jax.experimental.pallas import tpu as pltpu

EMB = 64
MAXN = 50


def _attn(nm, no, q, mask):
    scores = jnp.sum(nm * q[:, None, :], axis=-1)
    scores = jnp.where(mask, scores, jnp.finfo(scores.dtype).min)
    m = jnp.max(scores, axis=1, keepdims=True)
    e = jnp.exp(scores - m)
    p = e / jnp.sum(e, axis=1, keepdims=True)
    return jnp.sum(no * p[:, :, None], axis=1)


def _branch(u, v, nm, no, mask, hop_wT, hop_b, dense_wT, dense_b, out_w):
    q = u + v
    o0 = _attn(nm, no, q, mask)
    q1 = jax.nn.relu(
        jnp.dot(q, hop_wT, preferred_element_type=jnp.float32) + o0 + hop_b)
    o1 = _attn(nm, no, q1, mask)
    x = jnp.concatenate([u * v, o1], axis=1)
    h = jax.nn.relu(
        jnp.dot(x, dense_wT, preferred_element_type=jnp.float32) + dense_b)
    return jnp.sum(h * out_w, axis=1, keepdims=True)


def _cmn_kernel(u_ref, v_ref, vn_ref, nm_ref, no_ref, nmn_ref, non_ref,
                len_ref, lenn_ref, hop_wT_ref, hop_b_ref, dense_wT_ref,
                dense_b_ref, out_w_ref, pos_ref, neg_ref):
    u = u_ref[...]
    hop_wT = hop_wT_ref[...]
    hop_b = hop_b_ref[...]
    dense_wT = dense_wT_ref[...]
    dense_b = dense_b_ref[...]
    out_w = out_w_ref[...]
    bb = u.shape[0]
    pos_iota = jax.lax.broadcasted_iota(jnp.int32, (bb, MAXN), 1)
    mask = pos_iota < len_ref[...]
    mask_n = pos_iota < lenn_ref[...]
    pos_ref[...] = _branch(u, v_ref[...], nm_ref[...], no_ref[...], mask,
                           hop_wT, hop_b, dense_wT, dense_b, out_w)
    neg_ref[...] = _branch(u, vn_ref[...], nmn_ref[...], non_ref[...], mask_n,
                           hop_wT, hop_b, dense_wT, dense_b, out_w)


@functools.partial(jax.jit, static_argnames=("interpret",))
def _cmn_compute(cur_user, cur_item, cur_item_neg, nm, no, nmn, non,
                 lengths, lengths_n, hop_w, hop_b, dense_w, dense_b, out_w,
                 interpret=False):
    B = cur_user.shape[0]
    Bb = min(128, B)
    grid = (B // Bb,)
    row_spec = pl.BlockSpec((Bb, EMB), lambda i: (i, 0))
    neigh_spec = pl.BlockSpec((Bb, MAXN, EMB), lambda i: (i, 0, 0))
    len_spec = pl.BlockSpec((Bb, 1), lambda i: (i, 0))
    w_spec = pl.BlockSpec((EMB, EMB), lambda i: (0, 0))
    dw_spec = pl.BlockSpec((2 * EMB, EMB), lambda i: (0, 0))
    b_spec = pl.BlockSpec((1, EMB), lambda i: (0, 0))
    out_spec = pl.BlockSpec((Bb, 1), lambda i: (i, 0))
    pos, neg = pl.pallas_call(
        _cmn_kernel,
        grid=grid,
        in_specs=[row_spec, row_spec, row_spec,
                  neigh_spec, neigh_spec, neigh_spec, neigh_spec,
                  len_spec, len_spec, w_spec, b_spec, dw_spec, b_spec, b_spec],
        out_specs=[out_spec, out_spec],
        out_shape=[jax.ShapeDtypeStruct((B, 1), jnp.float32),
                   jax.ShapeDtypeStruct((B, 1), jnp.float32)],
        interpret=interpret,
    )(cur_user, cur_item, cur_item_neg, nm, no, nmn, non,
      lengths.reshape(B, 1), lengths_n.reshape(B, 1),
      hop_w.T, hop_b.reshape(1, EMB), dense_w.T, dense_b.reshape(1, EMB),
      out_w)
    return pos[:, 0], neg[:, 0]


def kernel(input_users, input_items, input_items_negative, input_neighborhoods,
           input_neighborhood_lengths, input_neighborhoods_negative,
           input_neighborhood_lengths_negative, user_memory, item_memory,
           user_output, hop_w, hop_b, dense_w, dense_b, out_w):
    cur_user = jnp.take(user_memory, input_users, axis=0)
    cur_item = jnp.take(item_memory, input_items, axis=0)
    cur_item_neg = jnp.take(item_memory, input_items_negative, axis=0)
    nm = jnp.take(user_memory, input_neighborhoods, axis=0)
    no = jnp.take(user_output, input_neighborhoods, axis=0)
    nmn = jnp.take(user_memory, input_neighborhoods_negative, axis=0)
    non = jnp.take(user_output, input_neighborhoods_negative, axis=0)
    return _cmn_compute(cur_user, cur_item, cur_item_neg, nm, no, nmn, non,
                        input_neighborhood_lengths,
                        input_neighborhood_lengths_negative,
                        hop_w, hop_b, dense_w, dense_b, out_w)


# speedup vs baseline: 1.8861x; 1.2597x over previous
import functools

import jax
import jax.numpy as jnp
from jax import lax
from jax.experimental import pallas as pl
from jax.experimental.pallas import tpu as pltpu
from jax.experimental.pallas import tpu_sc as plsc

EMB = 64
MAXN = 50
CH = 512
CHB = CH // 128


def _sc_gather_all(neigh_idx, neighn_idx, users_idx, items_idx, itemsn_idx,
                   user_memory, item_memory, user_output):
    info = plsc.get_sparse_core_info()
    NC, NS = info.num_cores, info.num_subcores
    NW = NC * NS

    n_neigh = neigh_idx.shape[0] * CH
    n_small = users_idx.shape[0] * CH

    out_types = [
        jax.ShapeDtypeStruct((n_neigh, EMB), jnp.float32),
        jax.ShapeDtypeStruct((n_neigh, EMB), jnp.float32),
        jax.ShapeDtypeStruct((n_neigh, EMB), jnp.float32),
        jax.ShapeDtypeStruct((n_neigh, EMB), jnp.float32),
        jax.ShapeDtypeStruct((n_small, EMB), jnp.float32),
        jax.ShapeDtypeStruct((n_small, EMB), jnp.float32),
        jax.ShapeDtypeStruct((n_small, EMB), jnp.float32),
    ]
    mesh = plsc.VectorSubcoreMesh(core_axis_name="c", subcore_axis_name="s")

    @functools.partial(
        pl.kernel, mesh=mesh, out_type=out_types,
        compiler_params=pltpu.CompilerParams(use_tc_tiling_on_sc=False),
        scratch_types=[
            pltpu.VMEM((CHB, 128), jnp.int32),
            pltpu.VMEM((CHB, 128), jnp.int32),
            pltpu.VMEM((CH, EMB), jnp.float32),
            pltpu.VMEM((CH, EMB), jnp.float32),
            pltpu.SemaphoreType.DMA,
            pltpu.SemaphoreType.DMA,
            pltpu.SemaphoreType.DMA,
            pltpu.SemaphoreType.DMA,
        ],
    )
    def sc_kernel(neigh_ref, neighn_ref, users_ref, items_ref, itemsn_ref,
                  umem_ref, imem_ref, uout_ref,
                  nm_ref, no_ref, nmn_ref, non_ref, cu_ref, ci_ref, cin_ref,
                  idx0, idx1, rows0, rows1, g0, g1, o0, o1):
        wid = lax.axis_index("s") * NC + lax.axis_index("c")
        idx_b = (idx0, idx1)
        rows_b = (rows0, rows1)
        g_sem = (g0, g1)
        o_sem = (o0, o1)

        def start(idx_src, table, chunk, b):
            pltpu.sync_copy(idx_src.at[chunk], idx_b[b])
            for j in range(CHB):
                pltpu.async_copy(table.at[idx_b[b].at[j]],
                                 rows_b[b].at[pl.ds(j * 128, 128)], g_sem[b])

        def finish(table, out, chunk, b):
            for j in range(CHB):
                pltpu.make_async_copy(table.at[idx_b[b].at[j]],
                                      rows_b[b].at[pl.ds(j * 128, 128)],
                                      g_sem[b]).wait()
            pltpu.async_copy(rows_b[b], out.at[pl.ds(chunk * CH, CH)], o_sem[b])

        def drain(out, chunk, b):
            pltpu.make_async_copy(rows_b[b], out.at[pl.ds(chunk * CH, CH)],
                                  o_sem[b]).wait()

        def run_task(idx_src, table, out):
            n_chunks = idx_src.shape[0] // NW
            c_lo = wid * n_chunks
            if n_chunks == 1:
                start(idx_src, table, c_lo, 0)
                finish(table, out, c_lo, 0)
                drain(out, c_lo, 0)
                return
            n_half = n_chunks // 2

            start(idx_src, table, c_lo, 0)

            def body(c2, _):
                c = c_lo + 2 * c2
                finish(table, out, c, 0)
                start(idx_src, table, c + 1, 1)
                drain(out, c, 0)
                finish(table, out, c + 1, 1)

                @pl.when(c2 + 1 < n_half)
                def _():
                    start(idx_src, table, c + 2, 0)

                drain(out, c + 1, 1)
                return ()

            lax.fori_loop(0, n_half, body, ())

        run_task(neigh_ref, umem_ref, nm_ref)
        run_task(neigh_ref, uout_ref, no_ref)
        run_task(neighn_ref, umem_ref, nmn_ref)
        run_task(neighn_ref, uout_ref, non_ref)
        run_task(users_ref, umem_ref, cu_ref)
        run_task(items_ref, imem_ref, ci_ref)
        run_task(itemsn_ref, imem_ref, cin_ref)

    return sc_kernel(neigh_idx, neighn_idx, users_idx, items_idx, itemsn_idx,
                     user_memory, item_memory, user_output)


def _attn(nm, no, q, mask):
    scores = jnp.sum(nm * q[:, None, :], axis=-1)
    scores = jnp.where(mask, scores, jnp.finfo(scores.dtype).min)
    m = jnp.max(scores, axis=1, keepdims=True)
    e = jnp.exp(scores - m)
    p = e / jnp.sum(e, axis=1, keepdims=True)
    return jnp.sum(no * p[:, :, None], axis=1)


def _bf16_dot(a, b):
    return jnp.dot(a.astype(jnp.bfloat16), b.astype(jnp.bfloat16),
                   preferred_element_type=jnp.float32)


def _branch(u, v, nm, no, mask, hop_wT, hop_b, dense_wT, dense_b, out_w):
    q = u + v
    o0 = _attn(nm, no, q, mask)
    q1 = jax.nn.relu(_bf16_dot(q, hop_wT) + o0 + hop_b)
    o1 = _attn(nm, no, q1, mask)
    x = jnp.concatenate([u * v, o1], axis=1)
    h = jax.nn.relu(_bf16_dot(x, dense_wT) + dense_b)
    hb = h.astype(jnp.bfloat16).astype(jnp.float32)
    wb = out_w.astype(jnp.bfloat16).astype(jnp.float32)
    return jnp.sum(hb * wb, axis=1, keepdims=True)


def _cmn_kernel(u_ref, v_ref, vn_ref, nm_ref, no_ref, nmn_ref, non_ref,
                len_ref, lenn_ref, hop_wT_ref, hop_b_ref, dense_wT_ref,
                dense_b_ref, out_w_ref, pos_ref, neg_ref):
    u = u_ref[...]
    hop_wT = hop_wT_ref[...]
    hop_b = hop_b_ref[...]
    dense_wT = dense_wT_ref[...]
    dense_b = dense_b_ref[...]
    out_w = out_w_ref[...]
    bb = u.shape[0]
    pos_iota = jax.lax.broadcasted_iota(jnp.int32, (bb, MAXN), 1)
    mask = pos_iota < len_ref[...]
    mask_n = pos_iota < lenn_ref[...]
    pos_ref[...] = _branch(u, v_ref[...], nm_ref[...], no_ref[...], mask,
                           hop_wT, hop_b, dense_wT, dense_b, out_w)
    neg_ref[...] = _branch(u, vn_ref[...], nmn_ref[...], non_ref[...], mask_n,
                           hop_wT, hop_b, dense_wT, dense_b, out_w)


def _cmn_compute(cur_user, cur_item, cur_item_neg, nm, no, nmn, non,
                 lengths, lengths_n, hop_w, hop_b, dense_w, dense_b, out_w,
                 interpret=False):
    B = cur_user.shape[0]
    Bb = min(128, B)
    grid = (B // Bb,)
    row_spec = pl.BlockSpec((Bb, EMB), lambda i: (i, 0))
    neigh_spec = pl.BlockSpec((Bb, MAXN, EMB), lambda i: (i, 0, 0))
    len_spec = pl.BlockSpec((Bb, 1), lambda i: (i, 0))
    w_spec = pl.BlockSpec((EMB, EMB), lambda i: (0, 0))
    dw_spec = pl.BlockSpec((2 * EMB, EMB), lambda i: (0, 0))
    b_spec = pl.BlockSpec((1, EMB), lambda i: (0, 0))
    out_spec = pl.BlockSpec((Bb, 1), lambda i: (i, 0))
    pos, neg = pl.pallas_call(
        _cmn_kernel,
        grid=grid,
        in_specs=[row_spec, row_spec, row_spec,
                  neigh_spec, neigh_spec, neigh_spec, neigh_spec,
                  len_spec, len_spec, w_spec, b_spec, dw_spec, b_spec, b_spec],
        out_specs=[out_spec, out_spec],
        out_shape=[jax.ShapeDtypeStruct((B, 1), jnp.float32),
                   jax.ShapeDtypeStruct((B, 1), jnp.float32)],
        interpret=interpret,
    )(cur_user, cur_item, cur_item_neg, nm, no, nmn, non,
      lengths.reshape(B, 1), lengths_n.reshape(B, 1),
      hop_w.T, hop_b.reshape(1, EMB), dense_w.T, dense_b.reshape(1, EMB),
      out_w)
    return pos[:, 0], neg[:, 0]


def kernel(input_users, input_items, input_items_negative, input_neighborhoods,
           input_neighborhood_lengths, input_neighborhoods_negative,
           input_neighborhood_lengths_negative, user_memory, item_memory,
           user_output, hop_w, hop_b, dense_w, dense_b, out_w):
    B = input_users.shape[0]
    nidx = input_neighborhoods.reshape(-1, CHB, 128)
    nnidx = input_neighborhoods_negative.reshape(-1, CHB, 128)
    uidx = input_users.reshape(-1, CHB, 128)
    iidx = input_items.reshape(-1, CHB, 128)
    inidx = input_items_negative.reshape(-1, CHB, 128)
    nm, no, nmn, non, cu, ci, cin = _sc_gather_all(
        nidx, nnidx, uidx, iidx, inidx, user_memory, item_memory, user_output)
    nm = nm.reshape(B, MAXN, EMB)
    no = no.reshape(B, MAXN, EMB)
    nmn = nmn.reshape(B, MAXN, EMB)
    non = non.reshape(B, MAXN, EMB)
    return _cmn_compute(cu, ci, cin, nm, no, nmn, non,
                        input_neighborhood_lengths,
                        input_neighborhood_lengths_negative,
                        hop_w, hop_b, dense_w, dense_b, out_w)
